# baseline (speedup 1.0000x reference)
import functools

import jax
import jax.numpy as jnp
from jax import lax
from jax.experimental import pallas as pl
from jax.experimental.pallas import tpu as pltpu
from jax.experimental.pallas import tpu_sc as plsc

B = 16384
D = 32
N = 1000000
NC = 2
NS = 16
NW = NC * NS
BPW = B // NW
L = 16
NGRP = BPW // L
EPW = D * BPW
CHUNK = 128
NCHUNK = EPW // CHUNK


def _sc_body(uid_hbm, iid_hbm, u_hbm, i_hbm, ub_hbm, ib_hbm,
             dots_hbm, bias_hbm,
             uidx, iidx, eu, ei, gu, gi, dv, ubv, ibv, bs, sem):
    wid = lax.axis_index("s") * NC + lax.axis_index("c")
    base = wid * BPW
    pltpu.sync_copy(uid_hbm.at[pl.ds(base, BPW)], uidx)
    pltpu.sync_copy(iid_hbm.at[pl.ds(base, BPW)], iidx)

    def idx_body(g, carry):
        sl = pl.ds(g * L, L)
        ru = uidx[sl]
        ri = iidx[sl]
        for d in range(D):
            dsl = pl.ds(d * BPW + g * L, L)
            eu[dsl] = ru + (d * N)
            ei[dsl] = ri + (d * N)
        return carry

    lax.fori_loop(0, NGRP, idx_body, 0)

    copies = []
    for c in range(NCHUNK):
        sl = pl.ds(c * CHUNK, CHUNK)
        copies.append(pltpu.async_copy(u_hbm.at[eu.at[sl]], gu.at[sl], sem))
        copies.append(pltpu.async_copy(i_hbm.at[ei.at[sl]], gi.at[sl], sem))
    for c in range(BPW // CHUNK):
        sl = pl.ds(c * CHUNK, CHUNK)
        copies.append(pltpu.async_copy(ub_hbm.at[uidx.at[sl]], ubv.at[sl], sem))
        copies.append(pltpu.async_copy(ib_hbm.at[iidx.at[sl]], ibv.at[sl], sem))
    for cp in copies:
        cp.wait()

    def mul_body(k, carry):
        sl = pl.ds(k * L, L)
        dv[sl] = gu[sl] * gi[sl]
        return carry

    lax.fori_loop(0, EPW // L, mul_body, 0)

    def bias_body(g, carry):
        sl = pl.ds(g * L, L)
        bs[sl] = ubv[sl] + ibv[sl]
        return carry

    lax.fori_loop(0, NGRP, bias_body, 0)

    out_copies = []
    for d in range(D):
        out_copies.append(pltpu.async_copy(
            dv.at[pl.ds(d * BPW, BPW)], dots_hbm.at[d, pl.ds(base, BPW)], sem))
    out_copies.append(pltpu.async_copy(bs, bias_hbm.at[pl.ds(base, BPW)], sem))
    for cp in out_copies:
        cp.wait()


@functools.cache
def _sc_gather():
    mesh = plsc.VectorSubcoreMesh(
        core_axis_name="c", subcore_axis_name="s", num_cores=NC, num_subcores=NS
    )
    return pl.kernel(
        _sc_body,
        out_type=(
            jax.ShapeDtypeStruct((D, B), jnp.float32),
            jax.ShapeDtypeStruct((B,), jnp.float32),
        ),
        mesh=mesh,
        compiler_params=pltpu.CompilerParams(
            use_tc_tiling_on_sc=False, needs_layout_passes=False
        ),
        scratch_types=(
            pltpu.VMEM((BPW,), jnp.int32),
            pltpu.VMEM((BPW,), jnp.int32),
            pltpu.VMEM((EPW,), jnp.int32),
            pltpu.VMEM((EPW,), jnp.int32),
            pltpu.VMEM((EPW,), jnp.float32),
            pltpu.VMEM((EPW,), jnp.float32),
            pltpu.VMEM((EPW,), jnp.float32),
            pltpu.VMEM((BPW,), jnp.float32),
            pltpu.VMEM((BPW,), jnp.float32),
            pltpu.VMEM((BPW,), jnp.float32),
            pltpu.SemaphoreType.DMA,
        ),
    )


def _tc_body(x_ref, bias_ref, w1_ref, b1_ref, w2_ref, b2_ref, o_ref):
    x = x_ref[...]
    h = jax.nn.sigmoid(
        jnp.dot(w1_ref[...], x, preferred_element_type=jnp.float32) + b1_ref[...]
    )
    z = jnp.dot(w2_ref[...], h, preferred_element_type=jnp.float32) + b2_ref[...]
    res = 1.0 + 0.5 * (jax.nn.sigmoid(z) - 0.5)
    o_ref[...] = jnp.sum(x * res, axis=0) + bias_ref[...]


_tc_mlp = pl.pallas_call(
    _tc_body,
    out_shape=jax.ShapeDtypeStruct((B,), jnp.float32),
)


def kernel(user_ids, item_ids, U, I, W1, b1, W2, b2, Ub, Ib):
    yu = U.T.reshape(-1)
    yi = I.T.reshape(-1)
    dots, bias = _sc_gather()(user_ids, item_ids, yu, yi,
                              Ub.reshape(-1), Ib.reshape(-1))
    return _tc_mlp(dots, bias, W1, b1.reshape(-1, 1), W2, b2.reshape(-1, 1))
